# baseline (speedup 1.0000x reference)
import jax
import jax.numpy as jnp
from jax.experimental import pallas as pl
from jax.experimental.pallas import tpu as pltpu

_N_ITEMS = 1_000_000
_D = 64
_BLOCK = 8_000
_GRID = _N_ITEMS // _BLOCK


def _score_body(uid_ref, user_ref, item_ref, out_ref):
    item_r = item_ref[...].astype(jnp.bfloat16).astype(jnp.float32)
    s = jax.lax.dot_general(
        item_r, user_ref[0, :, :],
        dimension_numbers=(((1,), (1,)), ((), ())),
        preferred_element_type=jnp.float32,
    )
    out_ref[0, :, :] = s


def _scores(user_id, user_emb, item_emb):
    uid = jnp.asarray(user_id, dtype=jnp.int32).reshape((1,))
    user3 = (user_emb.astype(jnp.bfloat16).astype(jnp.float32)
             .reshape((user_emb.shape[0], 1, _D)))
    grid_spec = pltpu.PrefetchScalarGridSpec(
        num_scalar_prefetch=1,
        grid=(_GRID,),
        in_specs=[
            pl.BlockSpec((1, 1, _D), lambda i, uid_ref: (uid_ref[0], 0, 0)),
            pl.BlockSpec((_BLOCK, _D), lambda i, uid_ref: (i, 0)),
        ],
        out_specs=pl.BlockSpec((1, _BLOCK, 1), lambda i, uid_ref: (i, 0, 0)),
    )
    out = pl.pallas_call(
        _score_body,
        grid_spec=grid_spec,
        out_shape=jax.ShapeDtypeStruct((_GRID, _BLOCK, 1), jnp.float32),
    )(uid, user3, item_emb)
    return out.reshape((_N_ITEMS,))


def kernel(user_id, user_emb, item_emb, topk):
    scores = _scores(user_id, user_emb, item_emb)
    _, idx = jax.lax.top_k(scores, 100)
    return idx.astype(jnp.int32)


# speedup vs baseline: 3.4916x; 3.4916x over previous
import jax
import jax.numpy as jnp
from jax.experimental import pallas as pl
from jax.experimental.pallas import tpu as pltpu

_N_ITEMS = 1_000_000
_D = 64
_BLOCK = 8_000
_GRID = _N_ITEMS // _BLOCK
_K = 100


def _score_body(uid_ref, user_ref, item_ref, out_ref):
    item_r = item_ref[...].astype(jnp.bfloat16).astype(jnp.float32)
    s = jax.lax.dot_general(
        user_ref[0, :, :], item_r,
        dimension_numbers=(((1,), (1,)), ((), ())),
        preferred_element_type=jnp.float32,
    )
    out_ref[0, :, :] = s


def _scores(user_id, user_emb, item_emb):
    uid = jnp.asarray(user_id, dtype=jnp.int32).reshape((1,))
    user3 = (user_emb.astype(jnp.bfloat16).astype(jnp.float32)
             .reshape((user_emb.shape[0], 1, _D)))
    grid_spec = pltpu.PrefetchScalarGridSpec(
        num_scalar_prefetch=1,
        grid=(_GRID,),
        in_specs=[
            pl.BlockSpec((1, 1, _D), lambda i, uid_ref: (uid_ref[0], 0, 0)),
            pl.BlockSpec((_BLOCK, _D), lambda i, uid_ref: (i, 0)),
        ],
        out_specs=pl.BlockSpec((1, 1, _BLOCK), lambda i, uid_ref: (i, 0, 0)),
    )
    return pl.pallas_call(
        _score_body,
        grid_spec=grid_spec,
        out_shape=jax.ShapeDtypeStruct((_GRID, 1, _BLOCK), jnp.float32),
    )(uid, user3, item_emb)


def _topk_body(s_ref, out_ref):
    neg_inf = jnp.float32(-jnp.inf)
    big = jnp.int32(2**30)
    iota_seg = jax.lax.broadcasted_iota(jnp.int32, (_GRID, 1), 0)
    iota_lane = jax.lax.broadcasted_iota(jnp.int32, (1, _BLOCK), 1)
    iota_out = jax.lax.broadcasted_iota(jnp.int32, (1, 128), 1)

    rm = jnp.max(s_ref[...], axis=2)

    def body(t, carry):
        rm, out_row = carry
        m = jnp.max(rm)
        seg = jnp.min(jnp.where(rm == m, iota_seg, big))
        row = s_ref[pl.ds(seg, 1), 0, :]
        lane = jnp.min(jnp.where(row == m, iota_lane, big))
        idx = seg * _BLOCK + lane
        newrow = jnp.where(iota_lane == lane, neg_inf, row)
        s_ref[pl.ds(seg, 1), 0, :] = newrow
        rm = jnp.where(iota_seg == seg, jnp.max(newrow), rm)
        out_row = jnp.where(iota_out == t, idx, out_row)
        return rm, out_row

    _, out_row = jax.lax.fori_loop(
        0, _K, body, (rm, jnp.zeros((1, 128), jnp.int32)))
    out_ref[...] = out_row


def _topk100(scores):
    out = pl.pallas_call(
        _topk_body,
        in_specs=[pl.BlockSpec((_GRID, 1, _BLOCK), lambda: (0, 0, 0))],
        out_specs=pl.BlockSpec((1, 128), lambda: (0, 0)),
        out_shape=jax.ShapeDtypeStruct((1, 128), jnp.int32),
    )(scores)
    return out[0, :_K]


def kernel(user_id, user_emb, item_emb, topk):
    scores = _scores(user_id, user_emb, item_emb)
    return _topk100(scores)
